# baseline (speedup 1.0000x reference)
import jax
import jax.numpy as jnp
from jax.experimental import pallas as pl
from jax.experimental.pallas import tpu as pltpu

_VMEM_LIMIT = 64 * 1024 * 1024


def _interp_matrix_t(n_in: int, n_out: int) -> jnp.ndarray:
    if n_out == 1 or n_in == 1:
        src = jnp.zeros((n_out,), dtype=jnp.float32)
    else:
        src = jnp.arange(n_out, dtype=jnp.float32) * ((n_in - 1) / (n_out - 1))
    i0 = jnp.clip(jnp.floor(src).astype(jnp.int32), 0, n_in - 1)
    i1 = jnp.clip(i0 + 1, 0, n_in - 1)
    frac = src - i0.astype(jnp.float32)
    m0 = jax.nn.one_hot(i0, n_in, dtype=jnp.float32) * (1.0 - frac)[:, None]
    m1 = jax.nn.one_hot(i1, n_in, dtype=jnp.float32) * frac[:, None]
    return (m0 + m1).T


def _up2x_packed_kernel(x_ref, ahte_ref, ahto_ref, awt_ref, o_ref):
    bc, h2, w2 = x_ref.shape
    w = w2 // 2
    h_out = ahte_ref.shape[1]
    w_out = awt_ref.shape[1]

    x2 = x_ref[...]
    lt = jnp.swapaxes(x2[:, :, :w], 1, 2)
    rt = jnp.swapaxes(x2[:, :, w:], 1, 2)

    v = jnp.dot(lt.reshape(bc * w, h2), ahte_ref[...],
                preferred_element_type=jnp.float32)
    v += jnp.dot(rt.reshape(bc * w, h2), ahto_ref[...],
                 preferred_element_type=jnp.float32)

    vt = jnp.swapaxes(v.reshape(bc, w, h_out), 1, 2)
    out = jnp.dot(vt.reshape(bc * h_out, w), awt_ref[...],
                  preferred_element_type=jnp.float32)
    o_ref[...] = out.reshape(bc, h_out, w_out)


def _up2x_kernel(x_ref, aht_ref, awt_ref, o_ref):
    bc, h, w = x_ref.shape
    h_out = aht_ref.shape[1]
    w_out = awt_ref.shape[1]
    xt = jnp.swapaxes(x_ref[...], 1, 2)
    v = jnp.dot(xt.reshape(bc * w, h), aht_ref[...],
                preferred_element_type=jnp.float32)
    vt = jnp.swapaxes(v.reshape(bc, w, h_out), 1, 2)
    out = jnp.dot(vt.reshape(bc * h_out, w), awt_ref[...],
                  preferred_element_type=jnp.float32)
    o_ref[...] = out.reshape(bc, h_out, w_out)


def kernel(x: jnp.ndarray) -> jnp.ndarray:
    n, c, h, w = x.shape
    h_out, w_out = 2 * h, 2 * w
    b = n * c

    a_h_t = _interp_matrix_t(h, h_out)
    a_w_t = _interp_matrix_t(w, w_out)

    bc = 128
    bc = max(1, min(bc, b))
    num_blocks = -(-b // bc)
    b_pad = num_blocks * bc

    packed = (2 * w == 128) and (h % 2 == 0)

    if packed:
        x_flat = x.reshape(b, h // 2, 2 * w)
    else:
        x_flat = x.reshape(b, h, w)
    if b_pad != b:
        x_flat = jnp.pad(x_flat, ((0, b_pad - b), (0, 0), (0, 0)))

    flops = 2 * b_pad * h * w * h_out + 2 * b_pad * h_out * w * w_out
    bytes_accessed = b_pad * (h * w + h_out * w_out) * 4

    if packed:
        ahte = a_h_t[0::2]
        ahto = a_h_t[1::2]
        out_flat = pl.pallas_call(
            _up2x_packed_kernel,
            out_shape=jax.ShapeDtypeStruct((b_pad, h_out, w_out), x.dtype),
            grid_spec=pltpu.PrefetchScalarGridSpec(
                num_scalar_prefetch=0,
                grid=(num_blocks,),
                in_specs=[
                    pl.BlockSpec((bc, h // 2, 2 * w), lambda i: (i, 0, 0)),
                    pl.BlockSpec((h // 2, h_out), lambda i: (0, 0)),
                    pl.BlockSpec((h // 2, h_out), lambda i: (0, 0)),
                    pl.BlockSpec((w, w_out), lambda i: (0, 0)),
                ],
                out_specs=pl.BlockSpec((bc, h_out, w_out), lambda i: (i, 0, 0)),
            ),
            compiler_params=pltpu.CompilerParams(
                dimension_semantics=("parallel",),
                vmem_limit_bytes=_VMEM_LIMIT),
            cost_estimate=pl.CostEstimate(
                flops=int(flops), transcendentals=0,
                bytes_accessed=int(bytes_accessed)),
        )(x_flat, ahte, ahto, a_w_t)
    else:
        out_flat = pl.pallas_call(
            _up2x_kernel,
            out_shape=jax.ShapeDtypeStruct((b_pad, h_out, w_out), x.dtype),
            grid_spec=pltpu.PrefetchScalarGridSpec(
                num_scalar_prefetch=0,
                grid=(num_blocks,),
                in_specs=[
                    pl.BlockSpec((bc, h, w), lambda i: (i, 0, 0)),
                    pl.BlockSpec((h, h_out), lambda i: (0, 0)),
                    pl.BlockSpec((w, w_out), lambda i: (0, 0)),
                ],
                out_specs=pl.BlockSpec((bc, h_out, w_out), lambda i: (i, 0, 0)),
            ),
            compiler_params=pltpu.CompilerParams(
                dimension_semantics=("parallel",),
                vmem_limit_bytes=_VMEM_LIMIT),
            cost_estimate=pl.CostEstimate(
                flops=int(flops), transcendentals=0,
                bytes_accessed=int(bytes_accessed)),
        )(x_flat, a_h_t, a_w_t)

    if b_pad != b:
        out_flat = out_flat[:b]
    return out_flat.reshape(n, c, h_out, w_out)


# speedup vs baseline: 1.0777x; 1.0777x over previous
import jax
import jax.numpy as jnp
from jax.experimental import pallas as pl
from jax.experimental.pallas import tpu as pltpu

_VMEM_LIMIT = 64 * 1024 * 1024


def _interp_matrix_t(n_in: int, n_out: int) -> jnp.ndarray:
    if n_out == 1 or n_in == 1:
        src = jnp.zeros((n_out,), dtype=jnp.float32)
    else:
        src = jnp.arange(n_out, dtype=jnp.float32) * ((n_in - 1) / (n_out - 1))
    i0 = jnp.clip(jnp.floor(src).astype(jnp.int32), 0, n_in - 1)
    i1 = jnp.clip(i0 + 1, 0, n_in - 1)
    frac = src - i0.astype(jnp.float32)
    m0 = jax.nn.one_hot(i0, n_in, dtype=jnp.float32) * (1.0 - frac)[:, None]
    m1 = jax.nn.one_hot(i1, n_in, dtype=jnp.float32) * frac[:, None]
    return (m0 + m1).T


def _up2x_packed_kernel(x_ref, awt2_ref, aht_ref, o_ref):
    bc, h2, w2 = x_ref.shape
    h = 2 * h2
    h_out = aht_ref.shape[1]
    w_out = w2

    up = jnp.dot(
        x_ref[...].reshape(bc * h2, w2), awt2_ref[...],
        preferred_element_type=jnp.float32,
    )

    u = up.reshape(bc, h, w_out)

    ut = jnp.swapaxes(u, 1, 2)
    v = jnp.dot(ut.reshape(bc * w_out, h), aht_ref[...],
                preferred_element_type=jnp.float32)
    o_ref[...] = jnp.swapaxes(v.reshape(bc, w_out, h_out), 1, 2)


def _up2x_kernel(x_ref, aht_ref, awt_ref, o_ref):
    bc, h, w = x_ref.shape
    h_out = aht_ref.shape[1]
    w_out = awt_ref.shape[1]
    xt = jnp.swapaxes(x_ref[...], 1, 2)
    v = jnp.dot(xt.reshape(bc * w, h), aht_ref[...],
                preferred_element_type=jnp.float32)
    vt = jnp.swapaxes(v.reshape(bc, w, h_out), 1, 2)
    out = jnp.dot(vt.reshape(bc * h_out, w), awt_ref[...],
                  preferred_element_type=jnp.float32)
    o_ref[...] = out.reshape(bc, h_out, w_out)


def kernel(x: jnp.ndarray) -> jnp.ndarray:
    n, c, h, w = x.shape
    h_out, w_out = 2 * h, 2 * w
    b = n * c

    a_h_t = _interp_matrix_t(h, h_out)
    a_w_t = _interp_matrix_t(w, w_out)

    bc = 128
    bc = max(1, min(bc, b))
    num_blocks = -(-b // bc)
    b_pad = num_blocks * bc

    packed = (2 * w == 128) and (h % 2 == 0)

    if packed:
        x_flat = x.reshape(b, h // 2, 2 * w)
    else:
        x_flat = x.reshape(b, h, w)
    if b_pad != b:
        x_flat = jnp.pad(x_flat, ((0, b_pad - b), (0, 0), (0, 0)))

    flops = 2 * b_pad * h * w * h_out + 2 * b_pad * h_out * w * w_out
    bytes_accessed = b_pad * (h * w + h_out * w_out) * 4

    if packed:
        awt2 = jax.scipy.linalg.block_diag(a_w_t, a_w_t)
        out_flat = pl.pallas_call(
            _up2x_packed_kernel,
            out_shape=jax.ShapeDtypeStruct((b_pad, h_out, w_out), x.dtype),
            grid_spec=pltpu.PrefetchScalarGridSpec(
                num_scalar_prefetch=0,
                grid=(num_blocks,),
                in_specs=[
                    pl.BlockSpec((bc, h // 2, 2 * w), lambda i: (i, 0, 0)),
                    pl.BlockSpec((2 * w, 4 * w), lambda i: (0, 0)),
                    pl.BlockSpec((h, h_out), lambda i: (0, 0)),
                ],
                out_specs=pl.BlockSpec((bc, h_out, w_out), lambda i: (i, 0, 0)),
            ),
            compiler_params=pltpu.CompilerParams(
                dimension_semantics=("parallel",),
                vmem_limit_bytes=_VMEM_LIMIT),
            cost_estimate=pl.CostEstimate(
                flops=int(flops), transcendentals=0,
                bytes_accessed=int(bytes_accessed)),
        )(x_flat, awt2, a_h_t)
    else:
        out_flat = pl.pallas_call(
            _up2x_kernel,
            out_shape=jax.ShapeDtypeStruct((b_pad, h_out, w_out), x.dtype),
            grid_spec=pltpu.PrefetchScalarGridSpec(
                num_scalar_prefetch=0,
                grid=(num_blocks,),
                in_specs=[
                    pl.BlockSpec((bc, h, w), lambda i: (i, 0, 0)),
                    pl.BlockSpec((h, h_out), lambda i: (0, 0)),
                    pl.BlockSpec((w, w_out), lambda i: (0, 0)),
                ],
                out_specs=pl.BlockSpec((bc, h_out, w_out), lambda i: (i, 0, 0)),
            ),
            compiler_params=pltpu.CompilerParams(
                dimension_semantics=("parallel",),
                vmem_limit_bytes=_VMEM_LIMIT),
            cost_estimate=pl.CostEstimate(
                flops=int(flops), transcendentals=0,
                bytes_accessed=int(bytes_accessed)),
        )(x_flat, a_h_t, a_w_t)

    if b_pad != b:
        out_flat = out_flat[:b]
    return out_flat.reshape(n, c, h_out, w_out)
